# baseline (speedup 1.0000x reference)
import functools

import jax
import jax.numpy as jnp
from jax import lax
from jax.experimental import pallas as pl
from jax.experimental.pallas import tpu as pltpu
from jax.experimental.pallas import tpu_sc as plsc

L = 16
NCLASS = 18
LANES = 16
TABLE_PAD = 384
NPARAM = L + L * (L + 1)
SCRATCH0 = 352
START0 = (L + 1) * (L + 1) + NCLASS + 1


def _log_sc(s):
    bits = plsc.bitcast(s, jnp.int32)
    y = (bits.astype(jnp.float32) - 1064866805.0) * 8.262958405176314e-08
    for _ in range(3):
        y = y - 1.0 + s * jnp.exp(-y)
    return y


def _make_sc_call(BS, T, n_workers):
    seq_per_w = BS // n_workers
    KT = T // 128
    P = BS * T
    KH = KT // 2
    n_blocks = seq_per_w * KH
    LAB_N = KT * 256
    mesh = plsc.VectorSubcoreMesh(core_axis_name="c", subcore_axis_name="s")

    @functools.partial(
        pl.kernel,
        out_type=jax.ShapeDtypeStruct((n_workers * LANES,), jnp.float32),
        mesh=mesh,
        scratch_types=[
            pltpu.VMEM((LAB_N + 256,), jnp.int32),
            pltpu.VMEM((n_blocks, 128), jnp.int32),
            pltpu.VMEM((n_blocks * 128,), jnp.float32),
            pltpu.VMEM((TABLE_PAD,), jnp.float32),
            pltpu.VMEM((NPARAM,), jnp.float32),
            pltpu.VMEM((LANES,), jnp.float32),
            pltpu.SemaphoreType.DMA,
            pltpu.SemaphoreType.DMA,
        ],
        compiler_params=pltpu.CompilerParams(needs_layout_passes=False),
    )
    def sc_call(lp_hbm, lab_hbm, asc_hbm, out_hbm,
                lab_v, idx_v, val_v, tab_v, raw_v, acc_v, sem_lab, sem_em):
        nc = mesh.num_cores
        wid = lax.axis_index("s") * nc + lax.axis_index("c")
        iota = lax.iota(jnp.int32, LANES)
        g = wid // 4
        r0 = (wid * seq_per_w) % 8

        pltpu.sync_copy(asc_hbm, raw_v)
        cols = [plsc.load_gather(raw_v, [iota * (L + 1) + (L + j)])
                for j in range(L + 1)]
        m = cols[0]
        for j in range(1, L + 1):
            m = jnp.maximum(m, cols[j])
        s = jnp.exp(cols[0] - m)
        for j in range(1, L + 1):
            s = s + jnp.exp(cols[j] - m)
        lse = m + _log_sc(s)
        for j in range(L + 1):
            plsc.store_scatter(
                tab_v, [iota * (L + 1) + (2 * (L + 1) + 2 + j)], cols[j] - lse)
        v0 = raw_v[pl.ds(0, LANES)]
        tab_v[pl.ds(SCRATCH0, LANES)] = plsc.cummax(v0)
        m0 = plsc.load_gather(
            tab_v, [jnp.full((LANES,), SCRATCH0 + LANES - 1, jnp.int32)])
        e0 = jnp.exp(v0 - m0)
        tab_v[pl.ds(SCRATCH0, LANES)] = plsc.cumsum(e0)
        s0 = plsc.load_gather(
            tab_v, [jnp.full((LANES,), SCRATCH0 + LANES - 1, jnp.int32)])
        tab_v[pl.ds(START0 + 2, LANES)] = v0 - m0 - _log_sc(s0)
        lab_src_base = g * (8 * T) + r0 * 128
        for k in range(KT):
            pltpu.async_copy(
                lab_hbm.at[pl.ds(lab_src_base + k * 1024, 256)],
                lab_v.at[pl.ds(k * 256, 256)], sem_lab)
        for j in range(seq_per_w):
            lab_v[pl.ds(LAB_N + j * 128, LANES)] = jnp.full(
                (LANES,), NCLASS, jnp.int32)
        pltpu.make_async_copy(
            lab_hbm.at[pl.ds(0, LAB_N)], lab_v.at[pl.ds(0, LAB_N)],
            sem_lab).wait()

        acc = jnp.zeros((LANES,), jnp.float32)
        zero = jnp.zeros((LANES,), jnp.float32)
        for j in range(seq_per_w):
            e_base = g * (8 * T) + (r0 + j) * 128

            def body(k, carry, gather, j=j, e_base=e_base):
                a0, a1 = carry
                lbase = k * 256 + j * 128
                for i in range(8):
                    off = i * LANES
                    prev = lab_v[pl.ds(lbase + off, LANES)]
                    if i < 7:
                        nxt = lab_v[pl.ds(lbase + off + 1, LANES)]
                    else:
                        tt = k * 128 + off + 1 + iota
                        nxt = plsc.load_gather(
                            lab_v,
                            [(tt >> 7) * 256 + j * 128 + (tt & 127)])
                    tv = plsc.load_gather(tab_v, [prev * (L + 1) + nxt])
                    if gather:
                        idx_v[j * KH + k, pl.ds(off, LANES)] = (
                            prev * P + (e_base + k * 1024 + off + iota))
                    if i % 2 == 0:
                        a0 = a0 + tv
                    else:
                        a1 = a1 + tv
                if gather:
                    pltpu.async_copy(
                        lp_hbm.at[idx_v.at[j * KH + k]],
                        val_v.at[pl.ds((j * KH + k) * 128, 128)], sem_em)
                return a0, a1

            acc, acc1 = lax.fori_loop(
                0, KH, lambda k, c: body(k, c, True), (acc, zero))
            acc, acc1 = lax.fori_loop(
                KH, KT, lambda k, c: body(k, c, False), (acc, acc1))
            acc = acc + acc1

            first = lab_v[pl.ds(j * 128, LANES)]
            sv = plsc.load_gather(tab_v, [START0 + first])
            acc = acc + jnp.where(iota == 0, sv, jnp.zeros_like(sv))

        pltpu.make_async_copy(
            lp_hbm.at[pl.ds(0, n_blocks * 128)], val_v, sem_em).wait()

        def red(n, c):
            b0, b1, b2, b3 = c
            rb = n * (4 * LANES)
            return (b0 + val_v[pl.ds(rb, LANES)],
                    b1 + val_v[pl.ds(rb + LANES, LANES)],
                    b2 + val_v[pl.ds(rb + 2 * LANES, LANES)],
                    b3 + val_v[pl.ds(rb + 3 * LANES, LANES)])
        b0, b1, b2, b3 = lax.fori_loop(
            0, n_blocks * 128 // (4 * LANES), red, (acc, zero, zero, zero))
        acc = (b0 + b1) + (b2 + b3)

        acc_v[...] = acc
        pltpu.sync_copy(acc_v, out_hbm.at[pl.ds(wid * LANES, LANES)])

    return sc_call


def _tc_half_emission(lp4, lab4, BS, T, C):
    KT = T // 128
    KH = KT // 2

    def tc_kernel(lp_ref, lab_ref, out_ref):
        @pl.when((pl.program_id(0) == 0) & (pl.program_id(1) == 0))
        def _():
            out_ref[0, 0] = 0.0
        lab = lab_ref[0, 0]
        x = jnp.where(lab == 2, lp_ref[2, 0, 0], 0.0)
        for c in range(3, C):
            x = x + jnp.where(lab == c, lp_ref[c, 0, 0], 0.0)
        out_ref[0, 0] += jnp.sum(x)

    out = pl.pallas_call(
        tc_kernel,
        grid=(BS // 8, KH),
        in_specs=[
            pl.BlockSpec((C, 1, 1, 8, 128),
                         lambda bg, kt: (0, bg, kt + KH, 0, 0)),
            pl.BlockSpec((1, 1, 8, 128),
                         lambda bg, kt: (bg, kt + KH, 0, 0)),
        ],
        out_specs=pl.BlockSpec(
            (1, 1), lambda bg, kt: (0, 0), memory_space=pltpu.SMEM),
        out_shape=jax.ShapeDtypeStruct((1, 1), jnp.float32),
    )(lp4, lab4)
    return out[0, 0]


def kernel(log_probs, input_lens, labels, A_scores):
    BS, T, C = log_probs.shape
    lp4 = (log_probs
           .transpose(2, 0, 1)
           .reshape(C, BS // 8, 8, T // 128, 128)
           .transpose(0, 1, 3, 2, 4))
    lab4 = (labels
            .reshape(BS // 8, 8, T // 128, 128)
            .transpose(0, 2, 1, 3))

    info = plsc.get_sparse_core_info()
    n_workers = info.num_cores * info.num_subcores
    sc_call = _make_sc_call(BS, T, n_workers)
    partials = sc_call(lp4.reshape(-1), lab4.reshape(-1), A_scores)
    em_hi = _tc_half_emission(lp4, lab4, BS, T, C)
    return (jnp.sum(partials) + em_hi) / (BS * T)


# speedup vs baseline: 3.6678x; 3.6678x over previous
import functools

import jax
import jax.numpy as jnp
from jax import lax
from jax.experimental import pallas as pl
from jax.experimental.pallas import tpu as pltpu
from jax.experimental.pallas import tpu_sc as plsc

L = 16
NCLASS = 18
LANES = 16
TABLE_PAD = 384
NPARAM = L + L * (L + 1)
SCRATCH0 = 352
START0 = (L + 1) * (L + 1) + NCLASS + 1


def _log_sc(s):
    bits = plsc.bitcast(s, jnp.int32)
    y = (bits.astype(jnp.float32) - 1064866805.0) * 8.262958405176314e-08
    for _ in range(3):
        y = y - 1.0 + s * jnp.exp(-y)
    return y


def _make_sc_call(BS, T, n_workers):
    seq_per_w = BS // n_workers
    KT = T // 128
    P = BS * T
    KH = KT // 2
    n_blocks = seq_per_w * KH
    LAB_N = KT * 256
    mesh = plsc.VectorSubcoreMesh(core_axis_name="c", subcore_axis_name="s")

    @functools.partial(
        pl.kernel,
        out_type=jax.ShapeDtypeStruct((n_workers * LANES,), jnp.float32),
        mesh=mesh,
        scratch_types=[
            pltpu.VMEM((LAB_N + 256,), jnp.int32),
            pltpu.VMEM((n_blocks, 128), jnp.int32),
            pltpu.VMEM((n_blocks * 128,), jnp.float32),
            pltpu.VMEM((TABLE_PAD,), jnp.float32),
            pltpu.VMEM((NPARAM,), jnp.float32),
            pltpu.VMEM((LANES,), jnp.float32),
            pltpu.SemaphoreType.DMA,
            pltpu.SemaphoreType.DMA,
        ],
        compiler_params=pltpu.CompilerParams(needs_layout_passes=False),
    )
    def sc_call(lp_hbm, lab_hbm, asc_hbm, out_hbm,
                lab_v, idx_v, val_v, tab_v, raw_v, acc_v, sem_lab, sem_em):
        nc = mesh.num_cores
        wid = lax.axis_index("s") * nc + lax.axis_index("c")
        iota = lax.iota(jnp.int32, LANES)
        g = wid // 4
        r0 = (wid * seq_per_w) % 8

        pltpu.sync_copy(asc_hbm, raw_v)
        cols = [plsc.load_gather(raw_v, [iota * (L + 1) + (L + j)])
                for j in range(L + 1)]
        m = cols[0]
        for j in range(1, L + 1):
            m = jnp.maximum(m, cols[j])
        s = jnp.exp(cols[0] - m)
        for j in range(1, L + 1):
            s = s + jnp.exp(cols[j] - m)
        lse = m + _log_sc(s)
        for j in range(L + 1):
            plsc.store_scatter(
                tab_v, [iota * (L + 1) + (2 * (L + 1) + 2 + j)], cols[j] - lse)
        v0 = raw_v[pl.ds(0, LANES)]
        tab_v[pl.ds(SCRATCH0, LANES)] = plsc.cummax(v0)
        m0 = plsc.load_gather(
            tab_v, [jnp.full((LANES,), SCRATCH0 + LANES - 1, jnp.int32)])
        e0 = jnp.exp(v0 - m0)
        tab_v[pl.ds(SCRATCH0, LANES)] = plsc.cumsum(e0)
        s0 = plsc.load_gather(
            tab_v, [jnp.full((LANES,), SCRATCH0 + LANES - 1, jnp.int32)])
        tab_v[pl.ds(START0 + 2, LANES)] = v0 - m0 - _log_sc(s0)
        lab_src_base = g * (8 * T) + r0 * 128
        for k in range(KT):
            pltpu.async_copy(
                lab_hbm.at[pl.ds(lab_src_base + k * 1024, 256)],
                lab_v.at[pl.ds(k * 256, 256)], sem_lab)
        for j in range(seq_per_w):
            lab_v[pl.ds(LAB_N + j * 128, LANES)] = jnp.full(
                (LANES,), NCLASS, jnp.int32)
        pltpu.make_async_copy(
            lab_hbm.at[pl.ds(0, LAB_N)], lab_v.at[pl.ds(0, LAB_N)],
            sem_lab).wait()

        acc = jnp.zeros((LANES,), jnp.float32)
        zero = jnp.zeros((LANES,), jnp.float32)
        for j in range(seq_per_w):
            e_base = g * (8 * T) + (r0 + j) * 128

            def body(k, carry, gather, j=j, e_base=e_base):
                a0, a1 = carry
                lbase = k * 256 + j * 128
                for i in range(8):
                    off = i * LANES
                    prev = lab_v[pl.ds(lbase + off, LANES)]
                    if i < 7:
                        nxt = lab_v[pl.ds(lbase + off + 1, LANES)]
                    else:
                        tt = k * 128 + off + 1 + iota
                        nxt = plsc.load_gather(
                            lab_v,
                            [(tt >> 7) * 256 + j * 128 + (tt & 127)])
                    tv = plsc.load_gather(tab_v, [prev * (L + 1) + nxt])
                    if gather:
                        idx_v[j * KH + k, pl.ds(off, LANES)] = (
                            prev * P + (e_base + k * 1024 + off + iota))
                    if i % 2 == 0:
                        a0 = a0 + tv
                    else:
                        a1 = a1 + tv
                if gather:
                    pltpu.async_copy(
                        lp_hbm.at[idx_v.at[j * KH + k]],
                        val_v.at[pl.ds((j * KH + k) * 128, 128)], sem_em)
                return a0, a1

            acc, acc1 = lax.fori_loop(
                0, KH, lambda k, c: body(k, c, True), (acc, zero))
            acc, acc1 = lax.fori_loop(
                KH, KT, lambda k, c: body(k, c, False), (acc, acc1))
            acc = acc + acc1

            first = lab_v[pl.ds(j * 128, LANES)]
            sv = plsc.load_gather(tab_v, [START0 + first])
            acc = acc + jnp.where(iota == 0, sv, jnp.zeros_like(sv))

        pltpu.make_async_copy(
            lp_hbm.at[pl.ds(0, n_blocks * 128)], val_v, sem_em).wait()

        def red(n, c):
            b0, b1, b2, b3 = c
            rb = n * (4 * LANES)
            return (b0 + val_v[pl.ds(rb, LANES)],
                    b1 + val_v[pl.ds(rb + LANES, LANES)],
                    b2 + val_v[pl.ds(rb + 2 * LANES, LANES)],
                    b3 + val_v[pl.ds(rb + 3 * LANES, LANES)])
        b0, b1, b2, b3 = lax.fori_loop(
            0, n_blocks * 128 // (4 * LANES), red, (acc, zero, zero, zero))
        acc = (b0 + b1) + (b2 + b3)

        acc_v[...] = acc
        pltpu.sync_copy(acc_v, out_hbm.at[pl.ds(wid * LANES, LANES)])

    return sc_call


def _tc_half_emission(lp4, lab4, BS, T, C):
    KT = T // 128
    KH = KT // 2

    KB = 8

    def tc_kernel(lp_ref, lab_ref, out_ref):
        @pl.when((pl.program_id(0) == 0) & (pl.program_id(1) == 0))
        def _():
            out_ref[...] = jnp.zeros_like(out_ref)
        lab = lab_ref[0]
        x = jnp.where(lab == 2, lp_ref[2, 0], 0.0)
        for c in range(3, C):
            x = x + jnp.where(lab == c, lp_ref[c, 0], 0.0)
        out_ref[...] += jnp.sum(x, axis=0)

    out = pl.pallas_call(
        tc_kernel,
        grid=(BS // 8, KH // KB),
        in_specs=[
            pl.BlockSpec((C, 1, KB, 8, 128),
                         lambda bg, kt: (0, bg, kt + KH // KB, 0, 0)),
            pl.BlockSpec((1, KB, 8, 128),
                         lambda bg, kt: (bg, kt + KH // KB, 0, 0)),
        ],
        out_specs=pl.BlockSpec(
            (8, 128), lambda bg, kt: (0, 0)),
        out_shape=jax.ShapeDtypeStruct((8, 128), jnp.float32),
    )(lp4, lab4)
    return jnp.sum(out)


def kernel(log_probs, input_lens, labels, A_scores):
    BS, T, C = log_probs.shape
    lp4 = (log_probs
           .transpose(2, 0, 1)
           .reshape(C, BS // 8, 8, T // 128, 128)
           .transpose(0, 1, 3, 2, 4))
    lab4 = (labels
            .reshape(BS // 8, 8, T // 128, 128)
            .transpose(0, 2, 1, 3))

    info = plsc.get_sparse_core_info()
    n_workers = info.num_cores * info.num_subcores
    sc_call = _make_sc_call(BS, T, n_workers)
    partials = sc_call(lp4.reshape(-1), lab4.reshape(-1), A_scores)
    em_hi = _tc_half_emission(lp4, lab4, BS, T, C)
    return (jnp.sum(partials) + em_hi) / (BS * T)
